# baseline (speedup 1.0000x reference)
import functools

import jax
import jax.numpy as jnp
from jax import lax
from jax.experimental import pallas as pl
from jax.experimental.pallas import tpu as pltpu
from jax.experimental.pallas import tpu_sc as plsc

N = 10000
E = 320000
D = 128

NC = 2
NS = 16
NW = NC * NS

NSLAB = 5
ES = E // NSLAB
EPW = ES // NW
CH = 128
NFULL = (EPW // CH) * CH
TAIL = EPW - NFULL

BE = 1280
GBS = ES // BE
EPS = 1e-5


def _vmesh():
    return plsc.VectorSubcoreMesh(core_axis_name="c", subcore_axis_name="s",
                                  num_cores=NC, num_subcores=NS)


NCH = (EPW + CH - 1) // CH


def _sc_gather(x, src, dst):
    @functools.partial(
        pl.kernel,
        out_type=(jax.ShapeDtypeStruct((ES, D), jnp.float32),
                  jax.ShapeDtypeStruct((ES, D), jnp.float32)),
        mesh=_vmesh(),
        scratch_types=[
            pltpu.VMEM((EPW,), jnp.int32), pltpu.VMEM((EPW,), jnp.int32),
            pltpu.VMEM((CH, D), jnp.float32), pltpu.VMEM((CH, D), jnp.float32),
            pltpu.VMEM((CH, D), jnp.float32), pltpu.VMEM((CH, D), jnp.float32),
            pltpu.SemaphoreType.DMA, pltpu.SemaphoreType.DMA,
            pltpu.SemaphoreType.DMA, pltpu.SemaphoreType.DMA,
        ],
    )
    def k(x_hbm, src_hbm, dst_hbm, xs_hbm, xd_hbm,
          si_all, di_all, sb0, sb1, db0, db1, gsem_s, gsem_d, ssem_s, ssem_d):
        wid = lax.axis_index("s") * NC + lax.axis_index("c")
        base = wid * EPW
        pltpu.sync_copy(src_hbm.at[pl.ds(base, EPW)], si_all)
        pltpu.sync_copy(dst_hbm.at[pl.ds(base, EPW)], di_all)
        sbufs = (sb0, sb1)
        dbufs = (db0, db1)
        s_store = [None, None]
        d_store = [None, None]
        for c in range(NCH):
            sz = CH if c < NCH - 1 else (EPW - (NCH - 1) * CH)
            off = c * CH
            bi = c % 2
            if s_store[bi] is not None:
                s_store[bi].wait()
                d_store[bi].wait()
            gs = pltpu.async_copy(x_hbm.at[si_all.at[pl.ds(off, sz)]],
                                  sbufs[bi].at[pl.ds(0, sz)], gsem_s)
            gd = pltpu.async_copy(x_hbm.at[di_all.at[pl.ds(off, sz)]],
                                  dbufs[bi].at[pl.ds(0, sz)], gsem_d)
            gs.wait()
            gd.wait()
            s_store[bi] = pltpu.async_copy(sbufs[bi].at[pl.ds(0, sz)],
                                           xs_hbm.at[pl.ds(base + off, sz)], ssem_s)
            d_store[bi] = pltpu.async_copy(dbufs[bi].at[pl.ds(0, sz)],
                                           xd_hbm.at[pl.ds(base + off, sz)], ssem_d)
        for bi in (0, 1):
            if s_store[bi] is not None:
                s_store[bi].wait()
                d_store[bi].wait()

    return k(x, src, dst)


def _silu(v):
    return v * jax.nn.sigmoid(v)


def _mlp_body(xd_ref, xs_ref, ea_ref, w1f_ref, w2f_ref, w1_ref, w2_ref,
              b1f_ref, b2f_ref, b1_ref, b2_ref, hf_ref, h_ref, st_ref):
    i = pl.program_id(0)
    xd = xd_ref[...].astype(jnp.bfloat16)
    xs = xs_ref[...].astype(jnp.bfloat16)
    ea = ea_ref[...].astype(jnp.bfloat16)
    w1f = w1f_ref[...]
    uf = (jnp.dot(xd, w1f[0:D], preferred_element_type=jnp.float32)
          + jnp.dot(xs, w1f[D:2 * D], preferred_element_type=jnp.float32)
          + jnp.dot(ea, w1f[2 * D:3 * D], preferred_element_type=jnp.float32)
          + b1f_ref[...])
    hf = jnp.dot(_silu(uf).astype(jnp.bfloat16), w2f_ref[...],
                 preferred_element_type=jnp.float32) + b2f_ref[...]
    w1 = w1_ref[...]
    u = (jnp.dot(xd, w1[0:D], preferred_element_type=jnp.float32)
         + jnp.dot(xs, w1[D:2 * D], preferred_element_type=jnp.float32)
         + jnp.dot(ea, w1[2 * D:3 * D], preferred_element_type=jnp.float32)
         + b1_ref[...])
    h = jnp.dot(_silu(u).astype(jnp.bfloat16), w2_ref[...],
                preferred_element_type=jnp.float32) + b2_ref[...]
    hf_ref[...] = hf.astype(jnp.bfloat16)
    h_ref[...] = h.astype(jnp.bfloat16)

    s1 = jnp.sum(hf, axis=0, keepdims=True)
    s2 = jnp.sum(hf * hf, axis=0, keepdims=True)
    upd = jnp.concatenate([s1, s2, jnp.zeros((6, D), jnp.float32)], axis=0)

    @pl.when(i == 0)
    def _():
        st_ref[...] = jnp.zeros_like(st_ref)

    st_ref[...] += upd


def _tc_mlp(xd, xs, ea, w1ft, w2ft, w1t, w2t, b1f, b2f, b1, b2):
    blk = lambda: pl.BlockSpec((BE, D), lambda i: (i, 0))
    full = lambda r: pl.BlockSpec((r, D), lambda i: (0, 0))
    return pl.pallas_call(
        _mlp_body,
        grid=(GBS,),
        in_specs=[blk(), blk(), blk(),
                  full(3 * D), full(D), full(3 * D), full(D),
                  full(1), full(1), full(1), full(1)],
        out_specs=[blk(), blk(), full(8)],
        out_shape=(jax.ShapeDtypeStruct((ES, D), jnp.bfloat16),
                   jax.ShapeDtypeStruct((ES, D), jnp.bfloat16),
                   jax.ShapeDtypeStruct((8, D), jnp.float32)),
    )(xd, xs, ea, w1ft, w2ft, w1t, w2t, b1f, b2f, b1, b2)


def _msg_body(hf_ref, h_ref, st_ref, gi_ref, bi_ref, msg_ref):
    st = jnp.sum(st_ref[...], axis=0)
    mean = st[0:1] * (1.0 / E)
    var = st[1:2] * (1.0 / E) - mean * mean
    inv = lax.rsqrt(var + EPS)
    hf = hf_ref[...].astype(jnp.float32)
    score = jax.nn.sigmoid((hf - mean) * inv * gi_ref[...] + bi_ref[...])
    msg_ref[...] = score * h_ref[...].astype(jnp.float32)


def _tc_msg(hf, h, st_all, gi, bi):
    blk = lambda: pl.BlockSpec((BE, D), lambda i: (i, 0))
    return pl.pallas_call(
        _msg_body,
        grid=(GBS,),
        in_specs=[blk(), blk(),
                  pl.BlockSpec((NSLAB, 8, D), lambda i: (0, 0, 0)),
                  pl.BlockSpec((1, D), lambda i: (0, 0)),
                  pl.BlockSpec((1, D), lambda i: (0, 0))],
        out_specs=blk(),
        out_shape=jax.ShapeDtypeStruct((ES, D), jnp.float32),
    )(hf, h, st_all, gi, bi)


def _sc_scatter(msg, dst, init0, init1):
    RPC = 624
    RTAIL = N - RPC * NS

    @functools.partial(
        pl.kernel,
        out_type=(jax.ShapeDtypeStruct((N, D), jnp.float32),
                  jax.ShapeDtypeStruct((N, D), jnp.float32)),
        mesh=_vmesh(),
        scratch_types=[
            pltpu.VMEM((CH,), jnp.int32), pltpu.VMEM((CH, D), jnp.float32),
            pltpu.VMEM((TAIL,), jnp.int32), pltpu.VMEM((TAIL, D), jnp.float32),
            pltpu.VMEM_SHARED((N, D), jnp.float32),
        ],
    )
    def k(msg_hbm, dst_hbm, z0_hbm, z1_hbm, o0_hbm, o1_hbm,
          idx, rows, idxt, rowst, acc):
        cid = lax.axis_index("c")
        sid = lax.axis_index("s")
        wid = sid * NC + cid
        base = wid * EPW

        @pl.when(cid == 0)
        def _():
            pltpu.sync_copy(z0_hbm.at[pl.ds(sid * RPC, RPC)], acc.at[pl.ds(sid * RPC, RPC)])

            @pl.when(sid == NS - 1)
            def _():
                pltpu.sync_copy(z0_hbm.at[pl.ds(NS * RPC, RTAIL)], acc.at[pl.ds(NS * RPC, RTAIL)])

        @pl.when(cid == 1)
        def _():
            pltpu.sync_copy(z1_hbm.at[pl.ds(sid * RPC, RPC)], acc.at[pl.ds(sid * RPC, RPC)])

            @pl.when(sid == NS - 1)
            def _():
                pltpu.sync_copy(z1_hbm.at[pl.ds(NS * RPC, RTAIL)], acc.at[pl.ds(NS * RPC, RTAIL)])

        plsc.subcore_barrier()

        @pl.loop(0, NFULL, step=CH)
        def _(off):
            b = base + off
            pltpu.sync_copy(dst_hbm.at[pl.ds(b, CH)], idx)
            pltpu.sync_copy(msg_hbm.at[pl.ds(b, CH)], rows)
            pltpu.sync_copy(rows, acc.at[idx], add=True)

        bt = base + NFULL
        pltpu.sync_copy(dst_hbm.at[pl.ds(bt, TAIL)], idxt)
        pltpu.sync_copy(msg_hbm.at[pl.ds(bt, TAIL)], rowst)
        pltpu.sync_copy(rowst, acc.at[idxt], add=True)

        plsc.subcore_barrier()

        @pl.when(cid == 0)
        def _():
            pltpu.sync_copy(acc.at[pl.ds(sid * RPC, RPC)], o0_hbm.at[pl.ds(sid * RPC, RPC)])

            @pl.when(sid == NS - 1)
            def _():
                pltpu.sync_copy(acc.at[pl.ds(NS * RPC, RTAIL)], o0_hbm.at[pl.ds(NS * RPC, RTAIL)])

        @pl.when(cid == 1)
        def _():
            pltpu.sync_copy(acc.at[pl.ds(sid * RPC, RPC)], o1_hbm.at[pl.ds(sid * RPC, RPC)])

            @pl.when(sid == NS - 1)
            def _():
                pltpu.sync_copy(acc.at[pl.ds(NS * RPC, RTAIL)], o1_hbm.at[pl.ds(NS * RPC, RTAIL)])

    return k(msg, dst, init0, init1)


def _final_body(*refs):
    part_refs = refs[:-4]
    x_ref, g_ref, b_ref, y_ref = refs[-4:]
    o = part_refs[0][...]
    for r in part_refs[1:]:
        o = o + r[...]
    mean = jnp.mean(o, axis=0, keepdims=True)
    var = jnp.mean(o * o, axis=0, keepdims=True) - mean * mean
    inv = lax.rsqrt(var + EPS)
    y = x_ref[...] + (o - mean) * inv * g_ref[...] + b_ref[...]
    y_ref[...] = jnp.maximum(y, 0.0)


def _tc_final(partials, x, g, b):
    nd = lambda: pl.BlockSpec((N, D), lambda: (0, 0))
    return pl.pallas_call(
        _final_body,
        in_specs=[nd() for _ in partials] + [nd(),
                  pl.BlockSpec((1, D), lambda: (0, 0)),
                  pl.BlockSpec((1, D), lambda: (0, 0))],
        out_specs=nd(),
        out_shape=jax.ShapeDtypeStruct((N, D), jnp.float32),
    )(*partials, x, g, b)


def kernel(x, edge_index, edge_attr, W1f, b1f, W2f, b2f, W1, b1, W2, b2,
           gamma_i, beta_i, gamma, beta):
    bf = jnp.bfloat16
    src = edge_index[0]
    dst = edge_index[1]
    w1ft = W1f.T.astype(bf)
    w2ft = W2f.T.astype(bf)
    w1t = W1.T.astype(bf)
    w2t = W2.T.astype(bf)

    gathered = []
    for s in range(NSLAB):
        sl = slice(s * ES, (s + 1) * ES)
        gathered.append(_sc_gather(x, src[sl], dst[sl]))

    mlp_out = []
    for s in range(NSLAB):
        sl = slice(s * ES, (s + 1) * ES)
        xs_g, xd_g = gathered[s]
        mlp_out.append(_tc_mlp(xd_g, xs_g, edge_attr[sl],
                               w1ft, w2ft, w1t, w2t,
                               b1f[None, :], b2f[None, :], b1[None, :], b2[None, :]))

    st_all = jnp.stack([m[2] for m in mlp_out], axis=0)

    o0 = jnp.zeros((N, D), jnp.float32)
    o1 = jnp.zeros((N, D), jnp.float32)
    for s in range(NSLAB):
        sl = slice(s * ES, (s + 1) * ES)
        hf, h, _ = mlp_out[s]
        msg = _tc_msg(hf, h, st_all, gamma_i[None, :], beta_i[None, :])
        o0, o1 = _sc_scatter(msg, dst[sl], o0, o1)

    return _tc_final([o0, o1], x, gamma[None, :], beta[None, :])


# speedup vs baseline: 1.1989x; 1.1989x over previous
import functools

import jax
import jax.numpy as jnp
from jax import lax
from jax.experimental import pallas as pl
from jax.experimental.pallas import tpu as pltpu
from jax.experimental.pallas import tpu_sc as plsc

N = 10000
E = 320000
D = 128

NC = 2
NS = 16
NW = NC * NS

NSLAB = 5
ES = E // NSLAB
EPW = ES // NW
CH = 128
NFULL = (EPW // CH) * CH
TAIL = EPW - NFULL

BE = 1280
GBS = ES // BE
EPS = 1e-5


def _vmesh():
    return plsc.VectorSubcoreMesh(core_axis_name="c", subcore_axis_name="s",
                                  num_cores=NC, num_subcores=NS)


NCH = (EPW + CH - 1) // CH


def _sc_gather(x, src, dst):
    @functools.partial(
        pl.kernel,
        out_type=(jax.ShapeDtypeStruct((ES, D), jnp.float32),
                  jax.ShapeDtypeStruct((ES, D), jnp.float32)),
        mesh=_vmesh(),
        scratch_types=[
            pltpu.VMEM((EPW,), jnp.int32), pltpu.VMEM((EPW,), jnp.int32),
            pltpu.VMEM((CH, D), jnp.float32), pltpu.VMEM((CH, D), jnp.float32),
            pltpu.VMEM((CH, D), jnp.float32), pltpu.VMEM((CH, D), jnp.float32),
            pltpu.SemaphoreType.DMA, pltpu.SemaphoreType.DMA,
            pltpu.SemaphoreType.DMA, pltpu.SemaphoreType.DMA,
        ],
    )
    def k(x_hbm, src_hbm, dst_hbm, xs_hbm, xd_hbm,
          si_all, di_all, sb0, sb1, db0, db1, gsem_s, gsem_d, ssem_s, ssem_d):
        sid = lax.axis_index("s")
        wid = sid * NC + lax.axis_index("c")
        base = wid * EPW
        pltpu.sync_copy(src_hbm.at[pl.ds(base, EPW)], si_all)
        pltpu.sync_copy(dst_hbm.at[pl.ds(base, EPW)], di_all)
        sbufs = (sb0, sb1)
        dbufs = (db0, db1)
        s_store = [None, None]
        d_store = [None, None]
        for c in range(NCH):
            sz = CH if c < NCH - 1 else (EPW - (NCH - 1) * CH)
            off = c * CH
            bi = c % 2
            if s_store[bi] is not None:
                s_store[bi].wait()
                d_store[bi].wait()
            gs = pltpu.async_copy(x_hbm.at[si_all.at[pl.ds(off, sz)]],
                                  sbufs[bi].at[pl.ds(0, sz)], gsem_s)
            gd = pltpu.async_copy(x_hbm.at[di_all.at[pl.ds(off, sz)]],
                                  dbufs[bi].at[pl.ds(0, sz)], gsem_d)
            gs.wait()
            gd.wait()
            s_store[bi] = pltpu.async_copy(sbufs[bi].at[pl.ds(0, sz)],
                                           xs_hbm.at[pl.ds(base + off, sz)], ssem_s)
            d_store[bi] = pltpu.async_copy(dbufs[bi].at[pl.ds(0, sz)],
                                           xd_hbm.at[pl.ds(base + off, sz)], ssem_d)
        for bi in (0, 1):
            if s_store[bi] is not None:
                s_store[bi].wait()
                d_store[bi].wait()

    return k(x, src, dst)


def _silu(v):
    return v * jax.nn.sigmoid(v)


def _mlp_body(xd_ref, xs_ref, ea_ref, w1f_ref, w2f_ref, w1_ref, w2_ref,
              b1f_ref, b2f_ref, b1_ref, b2_ref, hf_ref, h_ref, st_ref):
    i = pl.program_id(0)
    xd = xd_ref[...].astype(jnp.bfloat16)
    xs = xs_ref[...].astype(jnp.bfloat16)
    ea = ea_ref[...].astype(jnp.bfloat16)
    w1f = w1f_ref[...]
    uf = (jnp.dot(xd, w1f[0:D], preferred_element_type=jnp.float32)
          + jnp.dot(xs, w1f[D:2 * D], preferred_element_type=jnp.float32)
          + jnp.dot(ea, w1f[2 * D:3 * D], preferred_element_type=jnp.float32)
          + b1f_ref[...])
    hf = jnp.dot(_silu(uf).astype(jnp.bfloat16), w2f_ref[...],
                 preferred_element_type=jnp.float32) + b2f_ref[...]
    w1 = w1_ref[...]
    u = (jnp.dot(xd, w1[0:D], preferred_element_type=jnp.float32)
         + jnp.dot(xs, w1[D:2 * D], preferred_element_type=jnp.float32)
         + jnp.dot(ea, w1[2 * D:3 * D], preferred_element_type=jnp.float32)
         + b1_ref[...])
    h = jnp.dot(_silu(u).astype(jnp.bfloat16), w2_ref[...],
                preferred_element_type=jnp.float32) + b2_ref[...]
    hf_ref[...] = hf.astype(jnp.bfloat16)
    h_ref[...] = h.astype(jnp.bfloat16)

    s1 = jnp.sum(hf, axis=0, keepdims=True)
    s2 = jnp.sum(hf * hf, axis=0, keepdims=True)
    upd = jnp.concatenate([s1, s2, jnp.zeros((6, D), jnp.float32)], axis=0)

    @pl.when(i == 0)
    def _():
        st_ref[...] = jnp.zeros_like(st_ref)

    st_ref[...] += upd


def _tc_mlp(s, xd, xs, ea, w1ft, w2ft, w1t, w2t, b1f, b2f, b1, b2):
    blk = lambda: pl.BlockSpec((BE, D), lambda i: (i, 0))
    eablk = pl.BlockSpec((BE, D), lambda i: (s * GBS + i, 0))
    full = lambda r: pl.BlockSpec((r, D), lambda i: (0, 0))
    return pl.pallas_call(
        _mlp_body,
        grid=(GBS,),
        in_specs=[blk(), blk(), eablk,
                  full(3 * D), full(D), full(3 * D), full(D),
                  full(1), full(1), full(1), full(1)],
        out_specs=[blk(), blk(), full(8)],
        out_shape=(jax.ShapeDtypeStruct((ES, D), jnp.bfloat16),
                   jax.ShapeDtypeStruct((ES, D), jnp.bfloat16),
                   jax.ShapeDtypeStruct((8, D), jnp.float32)),
    )(xd, xs, ea, w1ft, w2ft, w1t, w2t, b1f, b2f, b1, b2)


def _msg_body(hf_ref, h_ref, st_ref, gi_ref, bi_ref, msg_ref):
    st = jnp.sum(st_ref[...], axis=0)
    mean = st[0:1] * (1.0 / E)
    var = st[1:2] * (1.0 / E) - mean * mean
    inv = lax.rsqrt(var + EPS)
    hf = hf_ref[...].astype(jnp.float32)
    score = jax.nn.sigmoid((hf - mean) * inv * gi_ref[...] + bi_ref[...])
    msg_ref[...] = score * h_ref[...].astype(jnp.float32)


def _tc_msg(hf, h, st_all, gi, bi):
    blk = lambda: pl.BlockSpec((BE, D), lambda i: (i, 0))
    return pl.pallas_call(
        _msg_body,
        grid=(GBS,),
        in_specs=[blk(), blk(),
                  pl.BlockSpec((NSLAB, 8, D), lambda i: (0, 0, 0)),
                  pl.BlockSpec((1, D), lambda i: (0, 0)),
                  pl.BlockSpec((1, D), lambda i: (0, 0))],
        out_specs=blk(),
        out_shape=jax.ShapeDtypeStruct((ES, D), jnp.float32),
    )(hf, h, st_all, gi, bi)


def _sc_scatter(msg, dst, init0, init1):
    RPC = 624
    RTAIL = N - RPC * NS

    @functools.partial(
        pl.kernel,
        out_type=(jax.ShapeDtypeStruct((N, D), jnp.float32),
                  jax.ShapeDtypeStruct((N, D), jnp.float32)),
        mesh=_vmesh(),
        scratch_types=[
            pltpu.VMEM((NCH - 1, CH), jnp.int32),
            pltpu.VMEM((CH, D), jnp.float32), pltpu.VMEM((CH, D), jnp.float32),
            pltpu.VMEM((TAIL,), jnp.int32), pltpu.VMEM((TAIL, D), jnp.float32),
            pltpu.VMEM_SHARED((N, D), jnp.float32),
            pltpu.SemaphoreType.DMA, pltpu.SemaphoreType.DMA,
            pltpu.SemaphoreType.DMA,
        ],
    )
    def k(msg_hbm, dst_hbm, z0_hbm, z1_hbm, o0_hbm, o1_hbm,
          idx2d, mb0, mb1, idxt, rowst, acc, isem, msem, scsem):
        cid = lax.axis_index("c")
        sid = lax.axis_index("s")
        wid = sid * NC + cid
        base = wid * EPW

        @pl.when(cid == 0)
        def _():
            pltpu.sync_copy(z0_hbm.at[pl.ds(sid * RPC, RPC)], acc.at[pl.ds(sid * RPC, RPC)])

            @pl.when(sid == NS - 1)
            def _():
                pltpu.sync_copy(z0_hbm.at[pl.ds(NS * RPC, RTAIL)], acc.at[pl.ds(NS * RPC, RTAIL)])

        @pl.when(cid == 1)
        def _():
            pltpu.sync_copy(z1_hbm.at[pl.ds(sid * RPC, RPC)], acc.at[pl.ds(sid * RPC, RPC)])

            @pl.when(sid == NS - 1)
            def _():
                pltpu.sync_copy(z1_hbm.at[pl.ds(NS * RPC, RTAIL)], acc.at[pl.ds(NS * RPC, RTAIL)])

        plsc.subcore_barrier()

        NCHF = NCH - 1
        idx_cps = [pltpu.async_copy(dst_hbm.at[pl.ds(base + c * CH, CH)],
                                    idx2d.at[c], isem) for c in range(NCHF)]
        mbufs = (mb0, mb1)
        loads = [None, None]
        scs = [None, None]
        loads[0] = pltpu.async_copy(msg_hbm.at[pl.ds(base, CH)], mb0, msem)
        for cp in idx_cps:
            cp.wait()
        for c in range(NCHF):
            bi = c % 2
            bo = (c + 1) % 2
            if c + 1 < NCHF:
                if scs[bo] is not None:
                    scs[bo].wait()
                loads[bo] = pltpu.async_copy(
                    msg_hbm.at[pl.ds(base + (c + 1) * CH, CH)], mbufs[bo], msem)
            loads[bi].wait()
            scs[bi] = pltpu.async_copy(mbufs[bi], acc.at[idx2d.at[c]], scsem,
                                       add=True)
        for bi in (0, 1):
            if scs[bi] is not None:
                scs[bi].wait()

        bt = base + NCHF * CH
        pltpu.sync_copy(dst_hbm.at[pl.ds(bt, TAIL)], idxt)
        pltpu.sync_copy(msg_hbm.at[pl.ds(bt, TAIL)], rowst)
        pltpu.sync_copy(rowst, acc.at[idxt], add=True)

        plsc.subcore_barrier()

        @pl.when(cid == 0)
        def _():
            pltpu.sync_copy(acc.at[pl.ds(sid * RPC, RPC)], o0_hbm.at[pl.ds(sid * RPC, RPC)])

            @pl.when(sid == NS - 1)
            def _():
                pltpu.sync_copy(acc.at[pl.ds(NS * RPC, RTAIL)], o0_hbm.at[pl.ds(NS * RPC, RTAIL)])

        @pl.when(cid == 1)
        def _():
            pltpu.sync_copy(acc.at[pl.ds(sid * RPC, RPC)], o1_hbm.at[pl.ds(sid * RPC, RPC)])

            @pl.when(sid == NS - 1)
            def _():
                pltpu.sync_copy(acc.at[pl.ds(NS * RPC, RTAIL)], o1_hbm.at[pl.ds(NS * RPC, RTAIL)])

    return k(msg, dst, init0, init1)


def _final_body(*refs):
    part_refs = refs[:-4]
    x_ref, g_ref, b_ref, y_ref = refs[-4:]
    o = part_refs[0][...]
    for r in part_refs[1:]:
        o = o + r[...]
    mean = jnp.mean(o, axis=0, keepdims=True)
    var = jnp.mean(o * o, axis=0, keepdims=True) - mean * mean
    inv = lax.rsqrt(var + EPS)
    y = x_ref[...] + (o - mean) * inv * g_ref[...] + b_ref[...]
    y_ref[...] = jnp.maximum(y, 0.0)


def _tc_final(partials, x, g, b):
    nd = lambda: pl.BlockSpec((N, D), lambda: (0, 0))
    return pl.pallas_call(
        _final_body,
        in_specs=[nd() for _ in partials] + [nd(),
                  pl.BlockSpec((1, D), lambda: (0, 0)),
                  pl.BlockSpec((1, D), lambda: (0, 0))],
        out_specs=nd(),
        out_shape=jax.ShapeDtypeStruct((N, D), jnp.float32),
    )(*partials, x, g, b)


def kernel(x, edge_index, edge_attr, W1f, b1f, W2f, b2f, W1, b1, W2, b2,
           gamma_i, beta_i, gamma, beta):
    bf = jnp.bfloat16
    src = edge_index[0]
    dst = edge_index[1]
    w1ft = W1f.T.astype(bf)
    w2ft = W2f.T.astype(bf)
    w1t = W1.T.astype(bf)
    w2t = W2.T.astype(bf)

    gathered = []
    for s in range(NSLAB):
        sl = slice(s * ES, (s + 1) * ES)
        gathered.append(_sc_gather(x, src[sl], dst[sl]))

    mlp_out = []
    for s in range(NSLAB):
        xs_g, xd_g = gathered[s]
        mlp_out.append(_tc_mlp(s, xd_g, xs_g, edge_attr,
                               w1ft, w2ft, w1t, w2t,
                               b1f[None, :], b2f[None, :], b1[None, :], b2[None, :]))

    st_all = jnp.stack([m[2] for m in mlp_out], axis=0)

    o0 = jnp.zeros((N, D), jnp.float32)
    o1 = jnp.zeros((N, D), jnp.float32)
    for s in range(NSLAB):
        sl = slice(s * ES, (s + 1) * ES)
        hf, h, _ = mlp_out[s]
        msg = _tc_msg(hf, h, st_all, gamma_i[None, :], beta_i[None, :])
        o0, o1 = _sc_scatter(msg, dst[sl], o0, o1)

    return _tc_final([o0, o1], x, gamma[None, :], beta[None, :])


# speedup vs baseline: 1.2499x; 1.0425x over previous
import functools

import jax
import jax.numpy as jnp
from jax import lax
from jax.experimental import pallas as pl
from jax.experimental.pallas import tpu as pltpu
from jax.experimental.pallas import tpu_sc as plsc

N = 10000
E = 320000
D = 128

NC = 2
NS = 16
NW = NC * NS

SLABS = (32000, 64000, 64000, 64000, 64000, 32000)
NSLAB = len(SLABS)
CH = 128

BE = 2000
EPS = 1e-5


def _vmesh():
    return plsc.VectorSubcoreMesh(core_axis_name="c", subcore_axis_name="s",
                                  num_cores=NC, num_subcores=NS)


def _sc_gather(x, src, dst, ES):
    EPW = ES // NW
    NCH = (EPW + CH - 1) // CH

    @functools.partial(
        pl.kernel,
        out_type=(jax.ShapeDtypeStruct((ES, D), jnp.float32),
                  jax.ShapeDtypeStruct((ES, D), jnp.float32)),
        mesh=_vmesh(),
        scratch_types=[
            pltpu.VMEM((EPW,), jnp.int32), pltpu.VMEM((EPW,), jnp.int32),
            pltpu.VMEM((CH, D), jnp.float32), pltpu.VMEM((CH, D), jnp.float32),
            pltpu.VMEM((CH, D), jnp.float32), pltpu.VMEM((CH, D), jnp.float32),
            pltpu.SemaphoreType.DMA, pltpu.SemaphoreType.DMA,
            pltpu.SemaphoreType.DMA, pltpu.SemaphoreType.DMA,
        ],
    )
    def k(x_hbm, src_hbm, dst_hbm, xs_hbm, xd_hbm,
          si_all, di_all, sb0, sb1, db0, db1, gsem_s, gsem_d, ssem_s, ssem_d):
        sid = lax.axis_index("s")
        wid = sid * NC + lax.axis_index("c")
        base = wid * EPW
        pltpu.sync_copy(src_hbm.at[pl.ds(base, EPW)], si_all)
        pltpu.sync_copy(dst_hbm.at[pl.ds(base, EPW)], di_all)
        sbufs = (sb0, sb1)
        dbufs = (db0, db1)
        s_store = [None, None]
        d_store = [None, None]
        for c in range(NCH):
            sz = CH if c < NCH - 1 else (EPW - (NCH - 1) * CH)
            off = c * CH
            bi = c % 2
            if s_store[bi] is not None:
                s_store[bi].wait()
                d_store[bi].wait()
            gs = pltpu.async_copy(x_hbm.at[si_all.at[pl.ds(off, sz)]],
                                  sbufs[bi].at[pl.ds(0, sz)], gsem_s)
            gd = pltpu.async_copy(x_hbm.at[di_all.at[pl.ds(off, sz)]],
                                  dbufs[bi].at[pl.ds(0, sz)], gsem_d)
            gs.wait()
            gd.wait()
            s_store[bi] = pltpu.async_copy(sbufs[bi].at[pl.ds(0, sz)],
                                           xs_hbm.at[pl.ds(base + off, sz)], ssem_s)
            d_store[bi] = pltpu.async_copy(dbufs[bi].at[pl.ds(0, sz)],
                                           xd_hbm.at[pl.ds(base + off, sz)], ssem_d)
        for bi in (0, 1):
            if s_store[bi] is not None:
                s_store[bi].wait()
                d_store[bi].wait()

    return k(x, src, dst)


def _silu(v):
    return v * jax.nn.sigmoid(v)


def _mlp_body(xd_ref, xs_ref, ea_ref, w1f_ref, w2f_ref, w1_ref, w2_ref,
              b1f_ref, b2f_ref, b1_ref, b2_ref, hf_ref, h_ref, st_ref):
    i = pl.program_id(0)
    xd = xd_ref[...].astype(jnp.bfloat16)
    xs = xs_ref[...].astype(jnp.bfloat16)
    ea = ea_ref[...].astype(jnp.bfloat16)
    w1f = w1f_ref[...]
    uf = (jnp.dot(xd, w1f[0:D], preferred_element_type=jnp.float32)
          + jnp.dot(xs, w1f[D:2 * D], preferred_element_type=jnp.float32)
          + jnp.dot(ea, w1f[2 * D:3 * D], preferred_element_type=jnp.float32)
          + b1f_ref[...])
    hf = jnp.dot(_silu(uf).astype(jnp.bfloat16), w2f_ref[...],
                 preferred_element_type=jnp.float32) + b2f_ref[...]
    w1 = w1_ref[...]
    u = (jnp.dot(xd, w1[0:D], preferred_element_type=jnp.float32)
         + jnp.dot(xs, w1[D:2 * D], preferred_element_type=jnp.float32)
         + jnp.dot(ea, w1[2 * D:3 * D], preferred_element_type=jnp.float32)
         + b1_ref[...])
    h = jnp.dot(_silu(u).astype(jnp.bfloat16), w2_ref[...],
                preferred_element_type=jnp.float32) + b2_ref[...]
    hf_ref[...] = hf.astype(jnp.bfloat16)
    h_ref[...] = h.astype(jnp.bfloat16)

    s1 = jnp.sum(hf, axis=0, keepdims=True)
    s2 = jnp.sum(hf * hf, axis=0, keepdims=True)
    upd = jnp.concatenate([s1, s2, jnp.zeros((6, D), jnp.float32)], axis=0)

    @pl.when(i == 0)
    def _():
        st_ref[...] = jnp.zeros_like(st_ref)

    st_ref[...] += upd


def _tc_mlp(e0, xd, xs, ea, w1ft, w2ft, w1t, w2t, b1f, b2f, b1, b2, ES):
    GBS = ES // BE
    b0 = e0 // BE
    blk = lambda: pl.BlockSpec((BE, D), lambda i: (i, 0))
    eablk = pl.BlockSpec((BE, D), lambda i: (b0 + i, 0))
    full = lambda r: pl.BlockSpec((r, D), lambda i: (0, 0))
    return pl.pallas_call(
        _mlp_body,
        grid=(GBS,),
        in_specs=[blk(), blk(), eablk,
                  full(3 * D), full(D), full(3 * D), full(D),
                  full(1), full(1), full(1), full(1)],
        out_specs=[blk(), blk(), full(8)],
        out_shape=(jax.ShapeDtypeStruct((ES, D), jnp.bfloat16),
                   jax.ShapeDtypeStruct((ES, D), jnp.bfloat16),
                   jax.ShapeDtypeStruct((8, D), jnp.float32)),
    )(xd, xs, ea, w1ft, w2ft, w1t, w2t, b1f, b2f, b1, b2)


def _msg_body(hf_ref, h_ref, st_ref, gi_ref, bi_ref, msg_ref):
    st = jnp.sum(st_ref[...], axis=0)
    mean = st[0:1] * (1.0 / E)
    var = st[1:2] * (1.0 / E) - mean * mean
    inv = lax.rsqrt(var + EPS)
    hf = hf_ref[...].astype(jnp.float32)
    score = jax.nn.sigmoid((hf - mean) * inv * gi_ref[...] + bi_ref[...])
    msg_ref[...] = score * h_ref[...].astype(jnp.float32)


def _tc_msg(hf, h, st_all, gi, bi, ES):
    GBS = ES // BE
    blk = lambda: pl.BlockSpec((BE, D), lambda i: (i, 0))
    return pl.pallas_call(
        _msg_body,
        grid=(GBS,),
        in_specs=[blk(), blk(),
                  pl.BlockSpec((NSLAB, 8, D), lambda i: (0, 0, 0)),
                  pl.BlockSpec((1, D), lambda i: (0, 0)),
                  pl.BlockSpec((1, D), lambda i: (0, 0))],
        out_specs=blk(),
        out_shape=jax.ShapeDtypeStruct((ES, D), jnp.float32),
    )(hf, h, st_all, gi, bi)


def _sc_scatter(msg, dst, init0, init1, ES):
    EPW = ES // NW
    NCH = (EPW + CH - 1) // CH
    TAIL = EPW - (NCH - 1) * CH
    RPC = 624
    RTAIL = N - RPC * NS

    @functools.partial(
        pl.kernel,
        out_type=(jax.ShapeDtypeStruct((N, D), jnp.float32),
                  jax.ShapeDtypeStruct((N, D), jnp.float32)),
        mesh=_vmesh(),
        scratch_types=[
            pltpu.VMEM((NCH - 1, CH), jnp.int32),
            pltpu.VMEM((CH, D), jnp.float32), pltpu.VMEM((CH, D), jnp.float32),
            pltpu.VMEM((TAIL,), jnp.int32), pltpu.VMEM((TAIL, D), jnp.float32),
            pltpu.VMEM_SHARED((N, D), jnp.float32),
            pltpu.SemaphoreType.DMA, pltpu.SemaphoreType.DMA,
            pltpu.SemaphoreType.DMA,
        ],
    )
    def k(msg_hbm, dst_hbm, z0_hbm, z1_hbm, o0_hbm, o1_hbm,
          idx2d, mb0, mb1, idxt, rowst, acc, isem, msem, scsem):
        cid = lax.axis_index("c")
        sid = lax.axis_index("s")
        wid = sid * NC + cid
        base = wid * EPW

        @pl.when(cid == 0)
        def _():
            pltpu.sync_copy(z0_hbm.at[pl.ds(sid * RPC, RPC)], acc.at[pl.ds(sid * RPC, RPC)])

            @pl.when(sid == NS - 1)
            def _():
                pltpu.sync_copy(z0_hbm.at[pl.ds(NS * RPC, RTAIL)], acc.at[pl.ds(NS * RPC, RTAIL)])

        @pl.when(cid == 1)
        def _():
            pltpu.sync_copy(z1_hbm.at[pl.ds(sid * RPC, RPC)], acc.at[pl.ds(sid * RPC, RPC)])

            @pl.when(sid == NS - 1)
            def _():
                pltpu.sync_copy(z1_hbm.at[pl.ds(NS * RPC, RTAIL)], acc.at[pl.ds(NS * RPC, RTAIL)])

        plsc.subcore_barrier()

        NCHF = NCH - 1
        idx_cps = [pltpu.async_copy(dst_hbm.at[pl.ds(base + c * CH, CH)],
                                    idx2d.at[c], isem) for c in range(NCHF)]
        mbufs = (mb0, mb1)
        loads = [None, None]
        scs = [None, None]
        loads[0] = pltpu.async_copy(msg_hbm.at[pl.ds(base, CH)], mb0, msem)
        for cp in idx_cps:
            cp.wait()
        for c in range(NCHF):
            bi = c % 2
            bo = (c + 1) % 2
            if c + 1 < NCHF:
                if scs[bo] is not None:
                    scs[bo].wait()
                loads[bo] = pltpu.async_copy(
                    msg_hbm.at[pl.ds(base + (c + 1) * CH, CH)], mbufs[bo], msem)
            loads[bi].wait()
            scs[bi] = pltpu.async_copy(mbufs[bi], acc.at[idx2d.at[c]], scsem,
                                       add=True)
        for bi in (0, 1):
            if scs[bi] is not None:
                scs[bi].wait()

        bt = base + NCHF * CH
        pltpu.sync_copy(dst_hbm.at[pl.ds(bt, TAIL)], idxt)
        pltpu.sync_copy(msg_hbm.at[pl.ds(bt, TAIL)], rowst)
        pltpu.sync_copy(rowst, acc.at[idxt], add=True)

        plsc.subcore_barrier()

        @pl.when(cid == 0)
        def _():
            pltpu.sync_copy(acc.at[pl.ds(sid * RPC, RPC)], o0_hbm.at[pl.ds(sid * RPC, RPC)])

            @pl.when(sid == NS - 1)
            def _():
                pltpu.sync_copy(acc.at[pl.ds(NS * RPC, RTAIL)], o0_hbm.at[pl.ds(NS * RPC, RTAIL)])

        @pl.when(cid == 1)
        def _():
            pltpu.sync_copy(acc.at[pl.ds(sid * RPC, RPC)], o1_hbm.at[pl.ds(sid * RPC, RPC)])

            @pl.when(sid == NS - 1)
            def _():
                pltpu.sync_copy(acc.at[pl.ds(NS * RPC, RTAIL)], o1_hbm.at[pl.ds(NS * RPC, RTAIL)])

    return k(msg, dst, init0, init1)


def _final_body(*refs):
    part_refs = refs[:-4]
    x_ref, g_ref, b_ref, y_ref = refs[-4:]
    o = part_refs[0][...]
    for r in part_refs[1:]:
        o = o + r[...]
    mean = jnp.mean(o, axis=0, keepdims=True)
    var = jnp.mean(o * o, axis=0, keepdims=True) - mean * mean
    inv = lax.rsqrt(var + EPS)
    y = x_ref[...] + (o - mean) * inv * g_ref[...] + b_ref[...]
    y_ref[...] = jnp.maximum(y, 0.0)


def _tc_final(partials, x, g, b):
    nd = lambda: pl.BlockSpec((N, D), lambda: (0, 0))
    return pl.pallas_call(
        _final_body,
        in_specs=[nd() for _ in partials] + [nd(),
                  pl.BlockSpec((1, D), lambda: (0, 0)),
                  pl.BlockSpec((1, D), lambda: (0, 0))],
        out_specs=nd(),
        out_shape=jax.ShapeDtypeStruct((N, D), jnp.float32),
    )(*partials, x, g, b)


def kernel(x, edge_index, edge_attr, W1f, b1f, W2f, b2f, W1, b1, W2, b2,
           gamma_i, beta_i, gamma, beta):
    bf = jnp.bfloat16
    src = edge_index[0]
    dst = edge_index[1]
    w1ft = W1f.T.astype(bf)
    w2ft = W2f.T.astype(bf)
    w1t = W1.T.astype(bf)
    w2t = W2.T.astype(bf)

    offs = [0]
    for es in SLABS:
        offs.append(offs[-1] + es)

    gathered = []
    for s in range(NSLAB):
        sl = slice(offs[s], offs[s + 1])
        gathered.append(_sc_gather(x, src[sl], dst[sl], SLABS[s]))

    mlp_out = []
    for s in range(NSLAB):
        xs_g, xd_g = gathered[s]
        mlp_out.append(_tc_mlp(offs[s], xd_g, xs_g, edge_attr,
                               w1ft, w2ft, w1t, w2t,
                               b1f[None, :], b2f[None, :], b1[None, :], b2[None, :],
                               SLABS[s]))

    st_all = jnp.stack([m[2] for m in mlp_out], axis=0)

    o0 = jnp.zeros((N, D), jnp.float32)
    o1 = jnp.zeros((N, D), jnp.float32)
    for s in range(NSLAB):
        sl = slice(offs[s], offs[s + 1])
        hf, h, _ = mlp_out[s]
        msg = _tc_msg(hf, h, st_all, gamma_i[None, :], beta_i[None, :], SLABS[s])
        o0, o1 = _sc_scatter(msg, dst[sl], o0, o1, SLABS[s])

    return _tc_final([o0, o1], x, gamma[None, :], beta[None, :])


# speedup vs baseline: 1.3561x; 1.0850x over previous
import functools

import jax
import jax.numpy as jnp
from jax import lax
from jax.experimental import pallas as pl
from jax.experimental.pallas import tpu as pltpu
from jax.experimental.pallas import tpu_sc as plsc

N = 10000
E = 320000
D = 128

NC = 2
NS = 16
NW = NC * NS

SLABS = (32000, 64000, 64000, 64000, 64000, 32000)
NSLAB = len(SLABS)
CH = 128

BE = 2000
EPS = 1e-5


def _vmesh():
    return plsc.VectorSubcoreMesh(core_axis_name="c", subcore_axis_name="s",
                                  num_cores=NC, num_subcores=NS)


def _sc_gather(ptab, qtab, src, dst, ES):
    EPW = ES // NW
    NCH = (EPW + CH - 1) // CH

    @functools.partial(
        pl.kernel,
        out_type=(jax.ShapeDtypeStruct((ES, D), jnp.int32),
                  jax.ShapeDtypeStruct((ES, D), jnp.int32)),
        mesh=_vmesh(),
        scratch_types=[
            pltpu.VMEM((EPW,), jnp.int32), pltpu.VMEM((EPW,), jnp.int32),
            pltpu.VMEM((CH, D), jnp.int32), pltpu.VMEM((CH, D), jnp.int32),
            pltpu.VMEM((CH, D), jnp.int32), pltpu.VMEM((CH, D), jnp.int32),
            pltpu.SemaphoreType.DMA, pltpu.SemaphoreType.DMA,
            pltpu.SemaphoreType.DMA, pltpu.SemaphoreType.DMA,
        ],
    )
    def k(p_hbm, q_hbm, src_hbm, dst_hbm, qs_hbm, pd_hbm,
          si_all, di_all, sb0, sb1, db0, db1, gsem_s, gsem_d, ssem_s, ssem_d):
        sid = lax.axis_index("s")
        wid = sid * NC + lax.axis_index("c")
        base = wid * EPW
        pltpu.sync_copy(src_hbm.at[pl.ds(base, EPW)], si_all)
        pltpu.sync_copy(dst_hbm.at[pl.ds(base, EPW)], di_all)
        sbufs = (sb0, sb1)
        dbufs = (db0, db1)
        s_store = [None, None]
        d_store = [None, None]
        for c in range(NCH):
            sz = CH if c < NCH - 1 else (EPW - (NCH - 1) * CH)
            off = c * CH
            bi = c % 2
            if s_store[bi] is not None:
                s_store[bi].wait()
                d_store[bi].wait()
            gs = pltpu.async_copy(q_hbm.at[si_all.at[pl.ds(off, sz)]],
                                  sbufs[bi].at[pl.ds(0, sz)], gsem_s)
            gd = pltpu.async_copy(p_hbm.at[di_all.at[pl.ds(off, sz)]],
                                  dbufs[bi].at[pl.ds(0, sz)], gsem_d)
            gs.wait()
            gd.wait()
            s_store[bi] = pltpu.async_copy(sbufs[bi].at[pl.ds(0, sz)],
                                           qs_hbm.at[pl.ds(base + off, sz)], ssem_s)
            d_store[bi] = pltpu.async_copy(dbufs[bi].at[pl.ds(0, sz)],
                                           pd_hbm.at[pl.ds(base + off, sz)], ssem_d)
        for bi in (0, 1):
            if s_store[bi] is not None:
                s_store[bi].wait()
                d_store[bi].wait()

    return k(ptab, qtab, src, dst)


def _pack_body(x_ref, w1f_ref, w1_ref, p_ref, q_ref):
    xb = x_ref[...].astype(jnp.bfloat16)
    w1f = w1f_ref[...]
    w1 = w1_ref[...]

    def pack(lo, hi):
        lo16 = jax.lax.bitcast_convert_type(lo.astype(jnp.bfloat16), jnp.uint16)
        hi16 = jax.lax.bitcast_convert_type(hi.astype(jnp.bfloat16), jnp.uint16)
        word = lo16.astype(jnp.uint32) | (hi16.astype(jnp.uint32) << 16)
        return jax.lax.bitcast_convert_type(word, jnp.int32)

    pf = jnp.dot(xb, w1f[0:D], preferred_element_type=jnp.float32)
    p2 = jnp.dot(xb, w1[0:D], preferred_element_type=jnp.float32)
    qf = jnp.dot(xb, w1f[D:2 * D], preferred_element_type=jnp.float32)
    q2 = jnp.dot(xb, w1[D:2 * D], preferred_element_type=jnp.float32)
    p_ref[...] = pack(pf, p2)
    q_ref[...] = pack(qf, q2)


def _tc_pack(x, w1ft, w1t):
    return pl.pallas_call(
        _pack_body,
        in_specs=[pl.BlockSpec((N, D), lambda: (0, 0)),
                  pl.BlockSpec((3 * D, D), lambda: (0, 0)),
                  pl.BlockSpec((3 * D, D), lambda: (0, 0))],
        out_specs=[pl.BlockSpec((N, D), lambda: (0, 0)),
                   pl.BlockSpec((N, D), lambda: (0, 0))],
        out_shape=(jax.ShapeDtypeStruct((N, D), jnp.int32),
                   jax.ShapeDtypeStruct((N, D), jnp.int32)),
    )(x, w1ft, w1t)


def _silu(v):
    return v * jax.nn.sigmoid(v)


def _unpack_lo(p):
    return jax.lax.bitcast_convert_type(jax.lax.shift_left(p, 16), jnp.float32)


def _unpack_hi(p):
    return jax.lax.bitcast_convert_type(
        jax.lax.bitwise_and(p, jnp.int32(-65536)), jnp.float32)


def _mlp_body(pd_ref, qs_ref, ea_ref, w1fc_ref, w2f_ref, w1c_ref, w2_ref,
              b1f_ref, b2f_ref, b1_ref, b2_ref, hf_ref, h_ref, st_ref):
    i = pl.program_id(0)
    pd = pd_ref[...]
    qs = qs_ref[...]
    ea = ea_ref[...].astype(jnp.bfloat16)
    uf = (_unpack_lo(pd) + _unpack_lo(qs)
          + jnp.dot(ea, w1fc_ref[...], preferred_element_type=jnp.float32)
          + b1f_ref[...])
    hf = jnp.dot(_silu(uf).astype(jnp.bfloat16), w2f_ref[...],
                 preferred_element_type=jnp.float32) + b2f_ref[...]
    u = (_unpack_hi(pd) + _unpack_hi(qs)
         + jnp.dot(ea, w1c_ref[...], preferred_element_type=jnp.float32)
         + b1_ref[...])
    h = jnp.dot(_silu(u).astype(jnp.bfloat16), w2_ref[...],
                preferred_element_type=jnp.float32) + b2_ref[...]
    hf_ref[...] = hf.astype(jnp.bfloat16)
    h_ref[...] = h.astype(jnp.bfloat16)

    s1 = jnp.sum(hf, axis=0, keepdims=True)
    s2 = jnp.sum(hf * hf, axis=0, keepdims=True)
    upd = jnp.concatenate([s1, s2, jnp.zeros((6, D), jnp.float32)], axis=0)

    @pl.when(i == 0)
    def _():
        st_ref[...] = jnp.zeros_like(st_ref)

    st_ref[...] += upd


def _tc_mlp(e0, xd, xs, ea, w1ft, w2ft, w1t, w2t, b1f, b2f, b1, b2, ES):
    GBS = ES // BE
    b0 = e0 // BE
    blk = lambda: pl.BlockSpec((BE, D), lambda i: (i, 0))
    eablk = pl.BlockSpec((BE, D), lambda i: (b0 + i, 0))
    full = lambda r: pl.BlockSpec((r, D), lambda i: (0, 0))
    return pl.pallas_call(
        _mlp_body,
        grid=(GBS,),
        in_specs=[blk(), blk(), eablk,
                  full(D), full(D), full(D), full(D),
                  full(1), full(1), full(1), full(1)],
        out_specs=[blk(), blk(), full(8)],
        out_shape=(jax.ShapeDtypeStruct((ES, D), jnp.bfloat16),
                   jax.ShapeDtypeStruct((ES, D), jnp.bfloat16),
                   jax.ShapeDtypeStruct((8, D), jnp.float32)),
    )(xd, xs, ea, w1ft, w2ft, w1t, w2t, b1f, b2f, b1, b2)


def _msg_body(hf_ref, h_ref, st_ref, gi_ref, bi_ref, msg_ref):
    st = jnp.sum(st_ref[...], axis=0)
    mean = st[0:1] * (1.0 / E)
    var = st[1:2] * (1.0 / E) - mean * mean
    inv = lax.rsqrt(var + EPS)
    hf = hf_ref[...].astype(jnp.float32)
    score = jax.nn.sigmoid((hf - mean) * inv * gi_ref[...] + bi_ref[...])
    msg_ref[...] = score * h_ref[...].astype(jnp.float32)


def _tc_msg(hf, h, st_all, gi, bi, ES):
    GBS = ES // BE
    blk = lambda: pl.BlockSpec((BE, D), lambda i: (i, 0))
    return pl.pallas_call(
        _msg_body,
        grid=(GBS,),
        in_specs=[blk(), blk(),
                  pl.BlockSpec((NSLAB, 8, D), lambda i: (0, 0, 0)),
                  pl.BlockSpec((1, D), lambda i: (0, 0)),
                  pl.BlockSpec((1, D), lambda i: (0, 0))],
        out_specs=blk(),
        out_shape=jax.ShapeDtypeStruct((ES, D), jnp.float32),
    )(hf, h, st_all, gi, bi)


def _sc_scatter(msg, dst, init0, init1, ES):
    EPW = ES // NW
    NCH = (EPW + CH - 1) // CH
    TAIL = EPW - (NCH - 1) * CH
    RPC = 624
    RTAIL = N - RPC * NS

    @functools.partial(
        pl.kernel,
        out_type=(jax.ShapeDtypeStruct((N, D), jnp.float32),
                  jax.ShapeDtypeStruct((N, D), jnp.float32)),
        mesh=_vmesh(),
        scratch_types=[
            pltpu.VMEM((NCH - 1, CH), jnp.int32),
            pltpu.VMEM((CH, D), jnp.float32), pltpu.VMEM((CH, D), jnp.float32),
            pltpu.VMEM((TAIL,), jnp.int32), pltpu.VMEM((TAIL, D), jnp.float32),
            pltpu.VMEM_SHARED((N, D), jnp.float32),
            pltpu.SemaphoreType.DMA, pltpu.SemaphoreType.DMA,
            pltpu.SemaphoreType.DMA,
        ],
    )
    def k(msg_hbm, dst_hbm, z0_hbm, z1_hbm, o0_hbm, o1_hbm,
          idx2d, mb0, mb1, idxt, rowst, acc, isem, msem, scsem):
        cid = lax.axis_index("c")
        sid = lax.axis_index("s")
        wid = sid * NC + cid
        base = wid * EPW

        @pl.when(cid == 0)
        def _():
            pltpu.sync_copy(z0_hbm.at[pl.ds(sid * RPC, RPC)], acc.at[pl.ds(sid * RPC, RPC)])

            @pl.when(sid == NS - 1)
            def _():
                pltpu.sync_copy(z0_hbm.at[pl.ds(NS * RPC, RTAIL)], acc.at[pl.ds(NS * RPC, RTAIL)])

        @pl.when(cid == 1)
        def _():
            pltpu.sync_copy(z1_hbm.at[pl.ds(sid * RPC, RPC)], acc.at[pl.ds(sid * RPC, RPC)])

            @pl.when(sid == NS - 1)
            def _():
                pltpu.sync_copy(z1_hbm.at[pl.ds(NS * RPC, RTAIL)], acc.at[pl.ds(NS * RPC, RTAIL)])

        plsc.subcore_barrier()

        NCHF = NCH - 1
        idx_cps = [pltpu.async_copy(dst_hbm.at[pl.ds(base + c * CH, CH)],
                                    idx2d.at[c], isem) for c in range(NCHF)]
        mbufs = (mb0, mb1)
        loads = [None, None]
        scs = [None, None]
        loads[0] = pltpu.async_copy(msg_hbm.at[pl.ds(base, CH)], mb0, msem)
        for cp in idx_cps:
            cp.wait()
        for c in range(NCHF):
            bi = c % 2
            bo = (c + 1) % 2
            if c + 1 < NCHF:
                if scs[bo] is not None:
                    scs[bo].wait()
                loads[bo] = pltpu.async_copy(
                    msg_hbm.at[pl.ds(base + (c + 1) * CH, CH)], mbufs[bo], msem)
            loads[bi].wait()
            scs[bi] = pltpu.async_copy(mbufs[bi], acc.at[idx2d.at[c]], scsem,
                                       add=True)
        for bi in (0, 1):
            if scs[bi] is not None:
                scs[bi].wait()

        bt = base + NCHF * CH
        pltpu.sync_copy(dst_hbm.at[pl.ds(bt, TAIL)], idxt)
        pltpu.sync_copy(msg_hbm.at[pl.ds(bt, TAIL)], rowst)
        pltpu.sync_copy(rowst, acc.at[idxt], add=True)

        plsc.subcore_barrier()

        @pl.when(cid == 0)
        def _():
            pltpu.sync_copy(acc.at[pl.ds(sid * RPC, RPC)], o0_hbm.at[pl.ds(sid * RPC, RPC)])

            @pl.when(sid == NS - 1)
            def _():
                pltpu.sync_copy(acc.at[pl.ds(NS * RPC, RTAIL)], o0_hbm.at[pl.ds(NS * RPC, RTAIL)])

        @pl.when(cid == 1)
        def _():
            pltpu.sync_copy(acc.at[pl.ds(sid * RPC, RPC)], o1_hbm.at[pl.ds(sid * RPC, RPC)])

            @pl.when(sid == NS - 1)
            def _():
                pltpu.sync_copy(acc.at[pl.ds(NS * RPC, RTAIL)], o1_hbm.at[pl.ds(NS * RPC, RTAIL)])

    return k(msg, dst, init0, init1)


def _final_body(*refs):
    part_refs = refs[:-4]
    x_ref, g_ref, b_ref, y_ref = refs[-4:]
    o = part_refs[0][...]
    for r in part_refs[1:]:
        o = o + r[...]
    mean = jnp.mean(o, axis=0, keepdims=True)
    var = jnp.mean(o * o, axis=0, keepdims=True) - mean * mean
    inv = lax.rsqrt(var + EPS)
    y = x_ref[...] + (o - mean) * inv * g_ref[...] + b_ref[...]
    y_ref[...] = jnp.maximum(y, 0.0)


def _tc_final(partials, x, g, b):
    nd = lambda: pl.BlockSpec((N, D), lambda: (0, 0))
    return pl.pallas_call(
        _final_body,
        in_specs=[nd() for _ in partials] + [nd(),
                  pl.BlockSpec((1, D), lambda: (0, 0)),
                  pl.BlockSpec((1, D), lambda: (0, 0))],
        out_specs=nd(),
        out_shape=jax.ShapeDtypeStruct((N, D), jnp.float32),
    )(*partials, x, g, b)


def kernel(x, edge_index, edge_attr, W1f, b1f, W2f, b2f, W1, b1, W2, b2,
           gamma_i, beta_i, gamma, beta):
    bf = jnp.bfloat16
    src = edge_index[0]
    dst = edge_index[1]
    w1ft = W1f.T.astype(bf)
    w2ft = W2f.T.astype(bf)
    w1t = W1.T.astype(bf)
    w2t = W2.T.astype(bf)
    w1fc = w1ft[2 * D:3 * D]
    w1c = w1t[2 * D:3 * D]
    ptab, qtab = _tc_pack(x, w1ft, w1t)

    offs = [0]
    for es in SLABS:
        offs.append(offs[-1] + es)

    gathered = []
    for s in range(NSLAB):
        sl = slice(offs[s], offs[s + 1])
        gathered.append(_sc_gather(ptab, qtab, src[sl], dst[sl], SLABS[s]))

    mlp_out = []
    for s in range(NSLAB):
        qs_g, pd_g = gathered[s]
        mlp_out.append(_tc_mlp(offs[s], pd_g, qs_g, edge_attr,
                               w1fc, w2ft, w1c, w2t,
                               b1f[None, :], b2f[None, :], b1[None, :], b2[None, :],
                               SLABS[s]))

    st_all = jnp.stack([m[2] for m in mlp_out], axis=0)

    o0 = jnp.zeros((N, D), jnp.float32)
    o1 = jnp.zeros((N, D), jnp.float32)
    for s in range(NSLAB):
        sl = slice(offs[s], offs[s + 1])
        hf, h, _ = mlp_out[s]
        msg = _tc_msg(hf, h, st_all, gamma_i[None, :], beta_i[None, :], SLABS[s])
        o0, o1 = _sc_scatter(msg, dst[sl], o0, o1, SLABS[s])

    return _tc_final([o0, o1], x, gamma[None, :], beta[None, :])


# speedup vs baseline: 1.4091x; 1.0391x over previous
import functools

import jax
import jax.numpy as jnp
from jax import lax
from jax.experimental import pallas as pl
from jax.experimental.pallas import tpu as pltpu
from jax.experimental.pallas import tpu_sc as plsc

N = 10000
E = 320000
D = 128

NC = 2
NS = 16
NW = NC * NS

SLABS = (32000, 64000, 64000, 64000, 64000, 32000)
NSLAB = len(SLABS)
CH = 128

BE = 4000
EPS = 1e-5


def _vmesh():
    return plsc.VectorSubcoreMesh(core_axis_name="c", subcore_axis_name="s",
                                  num_cores=NC, num_subcores=NS)


def _sc_gather(ptab, qtab, src, dst, ES):
    EPW = ES // NW
    NCH = (EPW + CH - 1) // CH

    @functools.partial(
        pl.kernel,
        out_type=(jax.ShapeDtypeStruct((ES, D), jnp.int32),
                  jax.ShapeDtypeStruct((ES, D), jnp.int32)),
        mesh=_vmesh(),
        scratch_types=[
            pltpu.VMEM((EPW,), jnp.int32), pltpu.VMEM((EPW,), jnp.int32),
            pltpu.VMEM((CH, D), jnp.int32), pltpu.VMEM((CH, D), jnp.int32),
            pltpu.VMEM((CH, D), jnp.int32), pltpu.VMEM((CH, D), jnp.int32),
            pltpu.SemaphoreType.DMA, pltpu.SemaphoreType.DMA,
            pltpu.SemaphoreType.DMA, pltpu.SemaphoreType.DMA,
        ],
    )
    def k(p_hbm, q_hbm, src_hbm, dst_hbm, qs_hbm, pd_hbm,
          si_all, di_all, sb0, sb1, db0, db1, gsem_s, gsem_d, ssem_s, ssem_d):
        sid = lax.axis_index("s")
        wid = sid * NC + lax.axis_index("c")
        base = wid * EPW
        pltpu.sync_copy(src_hbm.at[pl.ds(base, EPW)], si_all)
        pltpu.sync_copy(dst_hbm.at[pl.ds(base, EPW)], di_all)
        sbufs = (sb0, sb1)
        dbufs = (db0, db1)
        s_store = [None, None]
        d_store = [None, None]
        for c in range(NCH):
            sz = CH if c < NCH - 1 else (EPW - (NCH - 1) * CH)
            off = c * CH
            bi = c % 2
            if s_store[bi] is not None:
                s_store[bi].wait()
                d_store[bi].wait()
            gs = pltpu.async_copy(q_hbm.at[si_all.at[pl.ds(off, sz)]],
                                  sbufs[bi].at[pl.ds(0, sz)], gsem_s)
            gd = pltpu.async_copy(p_hbm.at[di_all.at[pl.ds(off, sz)]],
                                  dbufs[bi].at[pl.ds(0, sz)], gsem_d)
            gs.wait()
            gd.wait()
            s_store[bi] = pltpu.async_copy(sbufs[bi].at[pl.ds(0, sz)],
                                           qs_hbm.at[pl.ds(base + off, sz)], ssem_s)
            d_store[bi] = pltpu.async_copy(dbufs[bi].at[pl.ds(0, sz)],
                                           pd_hbm.at[pl.ds(base + off, sz)], ssem_d)
        for bi in (0, 1):
            if s_store[bi] is not None:
                s_store[bi].wait()
                d_store[bi].wait()

    return k(ptab, qtab, src, dst)


def _pack_body(x_ref, w1f_ref, w1_ref, p_ref, q_ref):
    xb = x_ref[...].astype(jnp.bfloat16)
    w1f = w1f_ref[...]
    w1 = w1_ref[...]

    def pack(lo, hi):
        lo16 = jax.lax.bitcast_convert_type(lo.astype(jnp.bfloat16), jnp.uint16)
        hi16 = jax.lax.bitcast_convert_type(hi.astype(jnp.bfloat16), jnp.uint16)
        word = lo16.astype(jnp.uint32) | (hi16.astype(jnp.uint32) << 16)
        return jax.lax.bitcast_convert_type(word, jnp.int32)

    pf = jnp.dot(xb, w1f[0:D], preferred_element_type=jnp.float32)
    p2 = jnp.dot(xb, w1[0:D], preferred_element_type=jnp.float32)
    qf = jnp.dot(xb, w1f[D:2 * D], preferred_element_type=jnp.float32)
    q2 = jnp.dot(xb, w1[D:2 * D], preferred_element_type=jnp.float32)
    p_ref[...] = pack(pf, p2)
    q_ref[...] = pack(qf, q2)


def _tc_pack(x, w1ft, w1t):
    return pl.pallas_call(
        _pack_body,
        in_specs=[pl.BlockSpec((N, D), lambda: (0, 0)),
                  pl.BlockSpec((3 * D, D), lambda: (0, 0)),
                  pl.BlockSpec((3 * D, D), lambda: (0, 0))],
        out_specs=[pl.BlockSpec((N, D), lambda: (0, 0)),
                   pl.BlockSpec((N, D), lambda: (0, 0))],
        out_shape=(jax.ShapeDtypeStruct((N, D), jnp.int32),
                   jax.ShapeDtypeStruct((N, D), jnp.int32)),
    )(x, w1ft, w1t)


def _silu(v):
    return v * jax.nn.sigmoid(v)


def _unpack_lo(p):
    return jax.lax.bitcast_convert_type(jax.lax.shift_left(p, 16), jnp.float32)


def _unpack_hi(p):
    return jax.lax.bitcast_convert_type(
        jax.lax.bitwise_and(p, jnp.int32(-65536)), jnp.float32)


def _mlp_body(pd_ref, qs_ref, ea_ref, w1fc_ref, w2f_ref, w1c_ref, w2_ref,
              b1f_ref, b2f_ref, b1_ref, b2_ref, hf_ref, h_ref, st_ref):
    i = pl.program_id(0)
    pd = pd_ref[...]
    qs = qs_ref[...]
    ea = ea_ref[...].astype(jnp.bfloat16)
    uf = (_unpack_lo(pd) + _unpack_lo(qs)
          + jnp.dot(ea, w1fc_ref[...], preferred_element_type=jnp.float32)
          + b1f_ref[...])
    hf = jnp.dot(_silu(uf).astype(jnp.bfloat16), w2f_ref[...],
                 preferred_element_type=jnp.float32) + b2f_ref[...]
    u = (_unpack_hi(pd) + _unpack_hi(qs)
         + jnp.dot(ea, w1c_ref[...], preferred_element_type=jnp.float32)
         + b1_ref[...])
    h = jnp.dot(_silu(u).astype(jnp.bfloat16), w2_ref[...],
                preferred_element_type=jnp.float32) + b2_ref[...]
    hf_ref[...] = hf.astype(jnp.bfloat16)
    h_ref[...] = h.astype(jnp.bfloat16)

    s1 = jnp.sum(hf, axis=0, keepdims=True)
    s2 = jnp.sum(hf * hf, axis=0, keepdims=True)
    upd = jnp.concatenate([s1, s2, jnp.zeros((6, D), jnp.float32)], axis=0)

    @pl.when(i == 0)
    def _():
        st_ref[...] = jnp.zeros_like(st_ref)

    st_ref[...] += upd


def _tc_mlp(e0, xd, xs, ea, w1ft, w2ft, w1t, w2t, b1f, b2f, b1, b2, ES):
    GBS = ES // BE
    b0 = e0 // BE
    blk = lambda: pl.BlockSpec((BE, D), lambda i: (i, 0))
    eablk = pl.BlockSpec((BE, D), lambda i: (b0 + i, 0))
    full = lambda r: pl.BlockSpec((r, D), lambda i: (0, 0))
    return pl.pallas_call(
        _mlp_body,
        grid=(GBS,),
        in_specs=[blk(), blk(), eablk,
                  full(D), full(D), full(D), full(D),
                  full(1), full(1), full(1), full(1)],
        out_specs=[blk(), blk(), full(8)],
        out_shape=(jax.ShapeDtypeStruct((ES, D), jnp.bfloat16),
                   jax.ShapeDtypeStruct((ES, D), jnp.bfloat16),
                   jax.ShapeDtypeStruct((8, D), jnp.float32)),
    )(xd, xs, ea, w1ft, w2ft, w1t, w2t, b1f, b2f, b1, b2)


def _msg_body(hf_ref, h_ref, st_ref, gi_ref, bi_ref, msg_ref):
    st = jnp.sum(st_ref[...], axis=0)
    mean = st[0:1] * (1.0 / E)
    var = st[1:2] * (1.0 / E) - mean * mean
    inv = lax.rsqrt(var + EPS)
    hf = hf_ref[...].astype(jnp.float32)
    score = jax.nn.sigmoid((hf - mean) * inv * gi_ref[...] + bi_ref[...])
    msg_ref[...] = score * h_ref[...].astype(jnp.float32)


def _tc_msg(hf, h, st_all, gi, bi, ES):
    GBS = ES // BE
    blk = lambda: pl.BlockSpec((BE, D), lambda i: (i, 0))
    return pl.pallas_call(
        _msg_body,
        grid=(GBS,),
        in_specs=[blk(), blk(),
                  pl.BlockSpec((NSLAB, 8, D), lambda i: (0, 0, 0)),
                  pl.BlockSpec((1, D), lambda i: (0, 0)),
                  pl.BlockSpec((1, D), lambda i: (0, 0))],
        out_specs=blk(),
        out_shape=jax.ShapeDtypeStruct((ES, D), jnp.float32),
    )(hf, h, st_all, gi, bi)


def _sc_scatter(msg, dst, init0, init1, ES):
    EPW = ES // NW
    NCH = (EPW + CH - 1) // CH
    TAIL = EPW - (NCH - 1) * CH
    RPC = 624
    RTAIL = N - RPC * NS

    @functools.partial(
        pl.kernel,
        out_type=(jax.ShapeDtypeStruct((N, D), jnp.float32),
                  jax.ShapeDtypeStruct((N, D), jnp.float32)),
        mesh=_vmesh(),
        scratch_types=[
            pltpu.VMEM((NCH - 1, CH), jnp.int32),
            pltpu.VMEM((CH, D), jnp.float32), pltpu.VMEM((CH, D), jnp.float32),
            pltpu.VMEM((TAIL,), jnp.int32), pltpu.VMEM((TAIL, D), jnp.float32),
            pltpu.VMEM_SHARED((N, D), jnp.float32),
            pltpu.SemaphoreType.DMA, pltpu.SemaphoreType.DMA,
            pltpu.SemaphoreType.DMA,
        ],
    )
    def k(msg_hbm, dst_hbm, z0_hbm, z1_hbm, o0_hbm, o1_hbm,
          idx2d, mb0, mb1, idxt, rowst, acc, isem, msem, scsem):
        cid = lax.axis_index("c")
        sid = lax.axis_index("s")
        wid = sid * NC + cid
        base = wid * EPW

        @pl.when(cid == 0)
        def _():
            pltpu.sync_copy(z0_hbm.at[pl.ds(sid * RPC, RPC)], acc.at[pl.ds(sid * RPC, RPC)])

            @pl.when(sid == NS - 1)
            def _():
                pltpu.sync_copy(z0_hbm.at[pl.ds(NS * RPC, RTAIL)], acc.at[pl.ds(NS * RPC, RTAIL)])

        @pl.when(cid == 1)
        def _():
            pltpu.sync_copy(z1_hbm.at[pl.ds(sid * RPC, RPC)], acc.at[pl.ds(sid * RPC, RPC)])

            @pl.when(sid == NS - 1)
            def _():
                pltpu.sync_copy(z1_hbm.at[pl.ds(NS * RPC, RTAIL)], acc.at[pl.ds(NS * RPC, RTAIL)])

        plsc.subcore_barrier()

        NCHF = NCH - 1
        idx_cps = [pltpu.async_copy(dst_hbm.at[pl.ds(base + c * CH, CH)],
                                    idx2d.at[c], isem) for c in range(NCHF)]
        mbufs = (mb0, mb1)
        loads = [None, None]
        scs = [None, None]
        loads[0] = pltpu.async_copy(msg_hbm.at[pl.ds(base, CH)], mb0, msem)
        for cp in idx_cps:
            cp.wait()
        for c in range(NCHF):
            bi = c % 2
            bo = (c + 1) % 2
            if c + 1 < NCHF:
                if scs[bo] is not None:
                    scs[bo].wait()
                loads[bo] = pltpu.async_copy(
                    msg_hbm.at[pl.ds(base + (c + 1) * CH, CH)], mbufs[bo], msem)
            loads[bi].wait()
            scs[bi] = pltpu.async_copy(mbufs[bi], acc.at[idx2d.at[c]], scsem,
                                       add=True)
        for bi in (0, 1):
            if scs[bi] is not None:
                scs[bi].wait()

        bt = base + NCHF * CH
        pltpu.sync_copy(dst_hbm.at[pl.ds(bt, TAIL)], idxt)
        pltpu.sync_copy(msg_hbm.at[pl.ds(bt, TAIL)], rowst)
        pltpu.sync_copy(rowst, acc.at[idxt], add=True)

        plsc.subcore_barrier()

        @pl.when(cid == 0)
        def _():
            pltpu.sync_copy(acc.at[pl.ds(sid * RPC, RPC)], o0_hbm.at[pl.ds(sid * RPC, RPC)])

            @pl.when(sid == NS - 1)
            def _():
                pltpu.sync_copy(acc.at[pl.ds(NS * RPC, RTAIL)], o0_hbm.at[pl.ds(NS * RPC, RTAIL)])

        @pl.when(cid == 1)
        def _():
            pltpu.sync_copy(acc.at[pl.ds(sid * RPC, RPC)], o1_hbm.at[pl.ds(sid * RPC, RPC)])

            @pl.when(sid == NS - 1)
            def _():
                pltpu.sync_copy(acc.at[pl.ds(NS * RPC, RTAIL)], o1_hbm.at[pl.ds(NS * RPC, RTAIL)])

    return k(msg, dst, init0, init1)


def _final_body(*refs):
    part_refs = refs[:-4]
    x_ref, g_ref, b_ref, y_ref = refs[-4:]
    o = part_refs[0][...]
    for r in part_refs[1:]:
        o = o + r[...]
    mean = jnp.mean(o, axis=0, keepdims=True)
    var = jnp.mean(o * o, axis=0, keepdims=True) - mean * mean
    inv = lax.rsqrt(var + EPS)
    y = x_ref[...] + (o - mean) * inv * g_ref[...] + b_ref[...]
    y_ref[...] = jnp.maximum(y, 0.0)


def _tc_final(partials, x, g, b):
    nd = lambda: pl.BlockSpec((N, D), lambda: (0, 0))
    return pl.pallas_call(
        _final_body,
        in_specs=[nd() for _ in partials] + [nd(),
                  pl.BlockSpec((1, D), lambda: (0, 0)),
                  pl.BlockSpec((1, D), lambda: (0, 0))],
        out_specs=nd(),
        out_shape=jax.ShapeDtypeStruct((N, D), jnp.float32),
    )(*partials, x, g, b)


def kernel(x, edge_index, edge_attr, W1f, b1f, W2f, b2f, W1, b1, W2, b2,
           gamma_i, beta_i, gamma, beta):
    bf = jnp.bfloat16
    src = edge_index[0]
    dst = edge_index[1]
    w1ft = W1f.T.astype(bf)
    w2ft = W2f.T.astype(bf)
    w1t = W1.T.astype(bf)
    w2t = W2.T.astype(bf)
    w1fc = w1ft[2 * D:3 * D]
    w1c = w1t[2 * D:3 * D]
    ptab, qtab = _tc_pack(x, w1ft, w1t)

    offs = [0]
    for es in SLABS:
        offs.append(offs[-1] + es)

    gathered = []
    for s in range(NSLAB):
        sl = slice(offs[s], offs[s + 1])
        gathered.append(_sc_gather(ptab, qtab, src[sl], dst[sl], SLABS[s]))

    mlp_out = []
    for s in range(NSLAB):
        qs_g, pd_g = gathered[s]
        mlp_out.append(_tc_mlp(offs[s], pd_g, qs_g, edge_attr,
                               w1fc, w2ft, w1c, w2t,
                               b1f[None, :], b2f[None, :], b1[None, :], b2[None, :],
                               SLABS[s]))

    st_all = jnp.stack([m[2] for m in mlp_out], axis=0)

    o0 = jnp.zeros((N, D), jnp.float32)
    o1 = jnp.zeros((N, D), jnp.float32)
    for s in range(NSLAB):
        sl = slice(offs[s], offs[s + 1])
        hf, h, _ = mlp_out[s]
        msg = _tc_msg(hf, h, st_all, gamma_i[None, :], beta_i[None, :], SLABS[s])
        o0, o1 = _sc_scatter(msg, dst[sl], o0, o1, SLABS[s])

    return _tc_final([o0, o1], x, gamma[None, :], beta[None, :])
